# baseline (speedup 1.0000x reference)
import functools

import jax
import jax.numpy as jnp
from jax import lax
from jax.experimental import pallas as pl
from jax.experimental.pallas import tpu as pltpu
from jax.experimental.pallas import tpu_sc as plsc

B = 32
N_EXAMPLES = 50000
N_CLASSES = 64
LANES = 16

SC_BATCHES = 8
WPB = B // SC_BATCHES
ROWS_PER_CHUNK = 500
CHUNK_ELEMS = ROWS_PER_CHUNK * N_CLASSES
CHUNKS_PER_BATCH = N_EXAMPLES // ROWS_PER_CHUNK
CHUNKS_PER_WORKER = CHUNKS_PER_BATCH // WPB
ROW_UNROLL = 4

TC_ROWS = 1000
SCALE = 1.0 / N_EXAMPLES


def _sc_body(x_hbm, out_hbm, buf0, buf1, hist, out_v, scr, sem0, sem1):
    wid = lax.axis_index("s") * 2 + lax.axis_index("c")
    batch = wid // WPB
    sub = wid % WPB
    chunk0 = sub * CHUNKS_PER_WORKER
    iota = lax.broadcasted_iota(jnp.int32, (LANES,), 0)
    zeros = jnp.zeros((LANES,), jnp.float32)
    ones = jnp.ones((LANES,), jnp.float32)
    lane0 = iota == 0
    xor_idx = [iota ^ sh for sh in (8, 4, 2, 1)]

    for j in range(N_CLASSES // LANES):
        hist[pl.ds(j * LANES, LANES)] = zeros

    def start(i, buf, sem):
        pltpu.async_copy(x_hbm.at[batch, chunk0 + i], buf, sem)

    def wait(buf, sem):
        pltpu.make_async_copy(x_hbm.at[0, 0], buf, sem).wait()

    def one_row(buf, r):
        off = r * N_CLASSES
        sbase = r * LANES
        q = [buf[pl.ds(off + k * LANES, LANES)] for k in range(4)]
        m = jnp.maximum(jnp.maximum(q[0], q[1]), jnp.maximum(q[2], q[3]))
        for p in xor_idx:
            scr[pl.ds(sbase, LANES)] = m
            t = plsc.load_gather(scr, [p + sbase])
            m = jnp.maximum(m, t)
        f = [plsc.all_reduce_ffs(q[k] == m) for k in range(4)]
        h = [jnp.where(f[k] == LANES, N_CLASSES, f[k] + k * LANES)
             for k in range(3)]
        cls = jnp.minimum(jnp.minimum(h[0], h[1]),
                          jnp.minimum(h[2], f[3] + 3 * LANES))
        cls = jnp.broadcast_to(cls, (LANES,)).astype(jnp.int32)
        plsc.addupdate_scatter(hist, [cls], ones, mask=lane0)

    def process(buf):
        @plsc.parallel_loop(0, ROWS_PER_CHUNK, unroll=ROW_UNROLL)
        def _rows(r):
            one_row(buf, r)

    C = CHUNKS_PER_WORKER
    start(0, buf0, sem0)
    if C % 2 == 1:
        def pair_body(k, _):
            start(2 * k + 1, buf1, sem1)
            wait(buf0, sem0)
            process(buf0)
            start(2 * k + 2, buf0, sem0)
            wait(buf1, sem1)
            process(buf1)
            return 0

        lax.fori_loop(0, (C - 1) // 2, pair_body, 0)
        wait(buf0, sem0)
        process(buf0)
    else:
        def pair_body(k, _):
            start(2 * k + 1, buf1, sem1)
            wait(buf0, sem0)
            process(buf0)
            start(2 * k + 2, buf0, sem0)
            wait(buf1, sem1)
            process(buf1)
            return 0

        lax.fori_loop(0, C // 2 - 1, pair_body, 0)
        start(C - 1, buf1, sem1)
        wait(buf0, sem0)
        process(buf0)
        wait(buf1, sem1)
        process(buf1)

    scale = jnp.float32(SCALE)
    for j in range(N_CLASSES // LANES):
        out_v[pl.ds(j * LANES, LANES)] = hist[pl.ds(j * LANES, LANES)] * scale

    pltpu.sync_copy(out_v, out_hbm.at[wid])


def _sc_run(x_sc):
    x3 = x_sc.reshape(SC_BATCHES, CHUNKS_PER_BATCH, CHUNK_ELEMS)
    mesh = plsc.VectorSubcoreMesh(core_axis_name="c", subcore_axis_name="s")
    k = functools.partial(
        pl.kernel,
        out_type=jax.ShapeDtypeStruct((B, N_CLASSES), jnp.float32),
        mesh=mesh,
        scratch_types=[
            pltpu.VMEM((CHUNK_ELEMS,), jnp.float32),
            pltpu.VMEM((CHUNK_ELEMS,), jnp.float32),
            pltpu.VMEM((N_CLASSES,), jnp.float32),
            pltpu.VMEM((N_CLASSES,), jnp.float32),
            pltpu.VMEM((ROWS_PER_CHUNK * LANES,), jnp.float32),
            pltpu.SemaphoreType.DMA,
            pltpu.SemaphoreType.DMA,
        ],
        compiler_params=pltpu.CompilerParams(needs_layout_passes=False),
    )(_sc_body)
    partial = k(x3)
    return partial.reshape(SC_BATCHES, WPB, N_CLASSES).sum(axis=1)


def _tc_body(x_ref, o_ref):
    j = pl.program_id(1)
    x = x_ref[0]
    iota2 = lax.broadcasted_iota(jnp.int32, (TC_ROWS, N_CLASSES), 1)
    m = jnp.max(x, axis=1, keepdims=True)
    amin = jnp.min(jnp.where(x == m, iota2, N_CLASSES), axis=1, keepdims=True)
    first = amin == iota2
    counts = jnp.sum(first.astype(jnp.float32), axis=0) * jnp.float32(SCALE)

    @pl.when(j == 0)
    def _():
        o_ref[...] = jnp.zeros_like(o_ref)

    o_ref[0, 0, :] += counts


def _tc_run(x_tc):
    bt = x_tc.shape[0]
    out = pl.pallas_call(
        _tc_body,
        grid=(bt, N_EXAMPLES // TC_ROWS),
        in_specs=[pl.BlockSpec((1, TC_ROWS, N_CLASSES),
                               lambda b, j: (b, j, 0))],
        out_specs=pl.BlockSpec((1, 1, N_CLASSES), lambda b, j: (b, 0, 0)),
        out_shape=jax.ShapeDtypeStruct((bt, 1, N_CLASSES), jnp.float32),
        compiler_params=pltpu.CompilerParams(
            dimension_semantics=("parallel", "arbitrary")),
    )(x_tc)
    return out.reshape(bt, N_CLASSES)


def kernel(input):
    sc_out = _sc_run(input[:SC_BATCHES])
    tc_out = _tc_run(input[SC_BATCHES:])
    return jnp.concatenate([sc_out, tc_out], axis=0)


# speedup vs baseline: 1.3659x; 1.3659x over previous
import functools

import jax
import jax.numpy as jnp
from jax import lax
from jax.experimental import pallas as pl
from jax.experimental.pallas import tpu as pltpu
from jax.experimental.pallas import tpu_sc as plsc

B = 32
N_EXAMPLES = 50000
N_CLASSES = 64
LANES = 16

SC_BATCHES = 8
WPB = B // SC_BATCHES
ROWS_PER_CHUNK = 500
CHUNK_ELEMS = ROWS_PER_CHUNK * N_CLASSES
CHUNKS_PER_BATCH = N_EXAMPLES // ROWS_PER_CHUNK
CHUNKS_PER_WORKER = CHUNKS_PER_BATCH // WPB
ROW_UNROLL = 4

TC_ROWS = 25000
SCALE = 1.0 / N_EXAMPLES


def _sc_body(x_hbm, out_hbm, buf0, buf1, hist, out_v, scr, sem0, sem1):
    wid = lax.axis_index("s") * 2 + lax.axis_index("c")
    batch = wid // WPB
    sub = wid % WPB
    chunk0 = sub * CHUNKS_PER_WORKER
    iota = lax.broadcasted_iota(jnp.int32, (LANES,), 0)
    zeros = jnp.zeros((LANES,), jnp.float32)
    ones = jnp.ones((LANES,), jnp.float32)
    lane0 = iota == 0
    xor_idx = [iota ^ sh for sh in (8, 4, 2, 1)]

    for j in range(N_CLASSES // LANES):
        hist[pl.ds(j * LANES, LANES)] = zeros

    def start(i, buf, sem):
        pltpu.async_copy(x_hbm.at[batch, chunk0 + i], buf, sem)

    def wait(buf, sem):
        pltpu.make_async_copy(x_hbm.at[0, 0], buf, sem).wait()

    def one_row(buf, r):
        off = r * N_CLASSES
        sbase = r * LANES
        q = [buf[pl.ds(off + k * LANES, LANES)] for k in range(4)]
        m = jnp.maximum(jnp.maximum(q[0], q[1]), jnp.maximum(q[2], q[3]))
        for p in xor_idx:
            scr[pl.ds(sbase, LANES)] = m
            t = plsc.load_gather(scr, [p + sbase])
            m = jnp.maximum(m, t)
        f = [plsc.all_reduce_ffs(q[k] == m) for k in range(4)]
        h = [jnp.where(f[k] == LANES, N_CLASSES, f[k] + k * LANES)
             for k in range(3)]
        cls = jnp.minimum(jnp.minimum(h[0], h[1]),
                          jnp.minimum(h[2], f[3] + 3 * LANES))
        cls = jnp.broadcast_to(cls, (LANES,)).astype(jnp.int32)
        plsc.addupdate_scatter(hist, [cls], ones, mask=lane0)

    def process(buf):
        @plsc.parallel_loop(0, ROWS_PER_CHUNK, unroll=ROW_UNROLL)
        def _rows(r):
            one_row(buf, r)

    C = CHUNKS_PER_WORKER
    start(0, buf0, sem0)
    if C % 2 == 1:
        def pair_body(k, _):
            start(2 * k + 1, buf1, sem1)
            wait(buf0, sem0)
            process(buf0)
            start(2 * k + 2, buf0, sem0)
            wait(buf1, sem1)
            process(buf1)
            return 0

        lax.fori_loop(0, (C - 1) // 2, pair_body, 0)
        wait(buf0, sem0)
        process(buf0)
    else:
        def pair_body(k, _):
            start(2 * k + 1, buf1, sem1)
            wait(buf0, sem0)
            process(buf0)
            start(2 * k + 2, buf0, sem0)
            wait(buf1, sem1)
            process(buf1)
            return 0

        lax.fori_loop(0, C // 2 - 1, pair_body, 0)
        start(C - 1, buf1, sem1)
        wait(buf0, sem0)
        process(buf0)
        wait(buf1, sem1)
        process(buf1)

    scale = jnp.float32(SCALE)
    for j in range(N_CLASSES // LANES):
        out_v[pl.ds(j * LANES, LANES)] = hist[pl.ds(j * LANES, LANES)] * scale

    pltpu.sync_copy(out_v, out_hbm.at[wid])


def _sc_run(x_sc):
    x3 = x_sc.reshape(SC_BATCHES, CHUNKS_PER_BATCH, CHUNK_ELEMS)
    mesh = plsc.VectorSubcoreMesh(core_axis_name="c", subcore_axis_name="s")
    k = functools.partial(
        pl.kernel,
        out_type=jax.ShapeDtypeStruct((B, N_CLASSES), jnp.float32),
        mesh=mesh,
        scratch_types=[
            pltpu.VMEM((CHUNK_ELEMS,), jnp.float32),
            pltpu.VMEM((CHUNK_ELEMS,), jnp.float32),
            pltpu.VMEM((N_CLASSES,), jnp.float32),
            pltpu.VMEM((N_CLASSES,), jnp.float32),
            pltpu.VMEM((ROWS_PER_CHUNK * LANES,), jnp.float32),
            pltpu.SemaphoreType.DMA,
            pltpu.SemaphoreType.DMA,
        ],
        compiler_params=pltpu.CompilerParams(needs_layout_passes=False),
    )(_sc_body)
    partial = k(x3)
    return partial.reshape(SC_BATCHES, WPB, N_CLASSES).sum(axis=1)


def _tc_body(x_ref, o_ref):
    j = pl.program_id(1)
    x = x_ref[0]
    iota2 = lax.broadcasted_iota(jnp.int32, (TC_ROWS, N_CLASSES), 1)
    m = jnp.max(x, axis=1, keepdims=True)
    amin = jnp.min(jnp.where(x == m, iota2, N_CLASSES), axis=1, keepdims=True)
    first = amin == iota2
    counts = jnp.sum(first.astype(jnp.float32), axis=0) * jnp.float32(SCALE)

    @pl.when(j == 0)
    def _():
        o_ref[...] = jnp.zeros_like(o_ref)

    o_ref[0, 0, :] += counts


def _tc_run(x_tc):
    bt = x_tc.shape[0]
    out = pl.pallas_call(
        _tc_body,
        grid=(bt, N_EXAMPLES // TC_ROWS),
        in_specs=[pl.BlockSpec((1, TC_ROWS, N_CLASSES),
                               lambda b, j: (b, j, 0))],
        out_specs=pl.BlockSpec((1, 1, N_CLASSES), lambda b, j: (b, 0, 0)),
        out_shape=jax.ShapeDtypeStruct((bt, 1, N_CLASSES), jnp.float32),
        compiler_params=pltpu.CompilerParams(
            dimension_semantics=("parallel", "arbitrary")),
    )(x_tc)
    return out.reshape(bt, N_CLASSES)


def kernel(input):
    sc_out = _sc_run(input[:SC_BATCHES])
    tc_out = _tc_run(input[SC_BATCHES:])
    return jnp.concatenate([sc_out, tc_out], axis=0)


# speedup vs baseline: 1.7101x; 1.2520x over previous
import functools

import jax
import jax.numpy as jnp
from jax import lax
from jax.experimental import pallas as pl
from jax.experimental.pallas import tpu as pltpu
from jax.experimental.pallas import tpu_sc as plsc

B = 32
N_EXAMPLES = 50000
N_CLASSES = 64
LANES = 16

SC_BATCHES = 8
WPB = B // SC_BATCHES
ROWS_PER_CHUNK = 500
CHUNK_ELEMS = ROWS_PER_CHUNK * N_CLASSES
CHUNKS_PER_BATCH = N_EXAMPLES // ROWS_PER_CHUNK
CHUNKS_PER_WORKER = CHUNKS_PER_BATCH // WPB
ROW_UNROLL = 4

TC_ROWS = 25000
SCALE = 1.0 / N_EXAMPLES


def _sc_body(x_hbm, out_hbm, buf0, buf1, hist, out_v, scr, sem0, sem1):
    wid = lax.axis_index("s") * 2 + lax.axis_index("c")
    batch = wid // WPB
    sub = wid % WPB
    chunk0 = sub * CHUNKS_PER_WORKER
    iota = lax.broadcasted_iota(jnp.int32, (LANES,), 0)
    zeros = jnp.zeros((LANES,), jnp.float32)
    ones = jnp.ones((LANES,), jnp.float32)
    lane0 = iota == 0
    xor_idx = [iota ^ sh for sh in (8, 4, 2, 1)]

    for j in range(N_CLASSES // LANES):
        hist[pl.ds(j * LANES, LANES)] = zeros

    def start(i, buf, sem):
        pltpu.async_copy(x_hbm.at[batch, chunk0 + i], buf, sem)

    def wait(buf, sem):
        pltpu.make_async_copy(x_hbm.at[0, 0], buf, sem).wait()

    def one_row(buf, r):
        off = r * N_CLASSES
        sbase = r * LANES
        q = [buf[pl.ds(off + k * LANES, LANES)] for k in range(4)]
        m = jnp.maximum(jnp.maximum(q[0], q[1]), jnp.maximum(q[2], q[3]))
        for p in xor_idx:
            scr[pl.ds(sbase, LANES)] = m
            t = plsc.load_gather(scr, [p + sbase])
            m = jnp.maximum(m, t)
        f = [plsc.all_reduce_ffs(q[k] == m) for k in range(4)]
        h = [jnp.where(f[k] == LANES, N_CLASSES, f[k] + k * LANES)
             for k in range(3)]
        cls = jnp.minimum(jnp.minimum(h[0], h[1]),
                          jnp.minimum(h[2], f[3] + 3 * LANES))
        cls = jnp.broadcast_to(cls, (LANES,)).astype(jnp.int32)
        plsc.addupdate_scatter(hist, [cls], ones, mask=lane0)

    def process(buf):
        @plsc.parallel_loop(0, ROWS_PER_CHUNK, unroll=ROW_UNROLL)
        def _rows(r):
            one_row(buf, r)

    C = CHUNKS_PER_WORKER
    start(0, buf0, sem0)
    if C % 2 == 1:
        def pair_body(k, _):
            start(2 * k + 1, buf1, sem1)
            wait(buf0, sem0)
            process(buf0)
            start(2 * k + 2, buf0, sem0)
            wait(buf1, sem1)
            process(buf1)
            return 0

        lax.fori_loop(0, (C - 1) // 2, pair_body, 0)
        wait(buf0, sem0)
        process(buf0)
    else:
        def pair_body(k, _):
            start(2 * k + 1, buf1, sem1)
            wait(buf0, sem0)
            process(buf0)
            start(2 * k + 2, buf0, sem0)
            wait(buf1, sem1)
            process(buf1)
            return 0

        lax.fori_loop(0, C // 2 - 1, pair_body, 0)
        start(C - 1, buf1, sem1)
        wait(buf0, sem0)
        process(buf0)
        wait(buf1, sem1)
        process(buf1)

    scale = jnp.float32(SCALE)
    for j in range(N_CLASSES // LANES):
        out_v[pl.ds(j * LANES, LANES)] = hist[pl.ds(j * LANES, LANES)] * scale

    pltpu.sync_copy(out_v, out_hbm.at[wid])


def _sc_run(x_sc):
    x3 = x_sc.reshape(SC_BATCHES, CHUNKS_PER_BATCH, CHUNK_ELEMS)
    mesh = plsc.VectorSubcoreMesh(core_axis_name="c", subcore_axis_name="s")
    k = functools.partial(
        pl.kernel,
        out_type=jax.ShapeDtypeStruct((B, N_CLASSES), jnp.float32),
        mesh=mesh,
        scratch_types=[
            pltpu.VMEM((CHUNK_ELEMS,), jnp.float32),
            pltpu.VMEM((CHUNK_ELEMS,), jnp.float32),
            pltpu.VMEM((N_CLASSES,), jnp.float32),
            pltpu.VMEM((N_CLASSES,), jnp.float32),
            pltpu.VMEM((ROWS_PER_CHUNK * LANES,), jnp.float32),
            pltpu.SemaphoreType.DMA,
            pltpu.SemaphoreType.DMA,
        ],
        compiler_params=pltpu.CompilerParams(needs_layout_passes=False),
    )(_sc_body)
    partial = k(x3)
    return partial.reshape(SC_BATCHES, WPB, N_CLASSES).sum(axis=1)


def _tc_body(x_ref, o_ref):
    j = pl.program_id(1)
    x = x_ref[0]
    iota2 = lax.broadcasted_iota(jnp.int32, (TC_ROWS, N_CLASSES), 1)
    amax = jnp.argmax(x, axis=1).astype(jnp.int32)
    first = amax[:, None] == iota2
    counts = jnp.sum(first.astype(jnp.float32), axis=0) * jnp.float32(SCALE)

    @pl.when(j == 0)
    def _():
        o_ref[...] = jnp.zeros_like(o_ref)

    o_ref[0, 0, :] += counts


def _tc_run(x_tc):
    bt = x_tc.shape[0]
    out = pl.pallas_call(
        _tc_body,
        grid=(bt, N_EXAMPLES // TC_ROWS),
        in_specs=[pl.BlockSpec((1, TC_ROWS, N_CLASSES),
                               lambda b, j: (b, j, 0))],
        out_specs=pl.BlockSpec((1, 1, N_CLASSES), lambda b, j: (b, 0, 0)),
        out_shape=jax.ShapeDtypeStruct((bt, 1, N_CLASSES), jnp.float32),
        compiler_params=pltpu.CompilerParams(
            dimension_semantics=("parallel", "arbitrary")),
    )(x_tc)
    return out.reshape(bt, N_CLASSES)


def kernel(input):
    sc_out = _sc_run(input[:SC_BATCHES])
    tc_out = _tc_run(input[SC_BATCHES:])
    return jnp.concatenate([sc_out, tc_out], axis=0)
